# Initial kernel scaffold; baseline (speedup 1.0000x reference)
import jax
import jax.numpy as jnp
from jax.experimental import pallas as pl


def kernel(teacher_input, W1, b1, W2, b2, edge_row, edge_col):
    raise NotImplementedError("write your pallas kernel here")


# speedup vs baseline: 15.1994x; 15.1994x over previous
import functools

import jax
import jax.numpy as jnp
from jax import lax
from jax.experimental import pallas as pl
from jax.experimental.pallas import tpu as pltpu
from jax.experimental.pallas import tpu_sc as plsc

N_USERS = 50000
N_ITEMS = 50000
N_EDGES = 800000
TEACHER_DIM = 256
HIDDEN = 128
EMB = 64

NC = 2
NS = 16
U_SC = N_USERS // NC
U_TILE = 1568
U_SC_PAD = NS * U_TILE
TRASH = U_SC
CHUNK = 128
E_PAD = N_EDGES + CHUNK
U_LAST = U_SC - (NS - 1) * U_TILE

MLP_BLOCK = 1000


def _mlp_body(x_ref, w1_ref, b1_ref, w2_ref, b2_ref, fn_ref):
    h = jnp.maximum(
        jnp.dot(x_ref[...], w1_ref[...], preferred_element_type=jnp.float32)
        + b1_ref[...], 0.0)
    f = jnp.dot(h, w2_ref[...], preferred_element_type=jnp.float32) + b2_ref[...]
    nrm = jnp.sqrt(jnp.sum(f * f, axis=-1, keepdims=True))
    fn_ref[...] = f / jnp.maximum(nrm, 1e-12)


def _mlp(x, W1, b1, W2, b2):
    return pl.pallas_call(
        _mlp_body,
        grid=(N_ITEMS // MLP_BLOCK,),
        in_specs=[
            pl.BlockSpec((MLP_BLOCK, TEACHER_DIM), lambda i: (i, 0)),
            pl.BlockSpec((TEACHER_DIM, HIDDEN), lambda i: (0, 0)),
            pl.BlockSpec((1, HIDDEN), lambda i: (0, 0)),
            pl.BlockSpec((HIDDEN, EMB), lambda i: (0, 0)),
            pl.BlockSpec((1, EMB), lambda i: (0, 0)),
        ],
        out_specs=pl.BlockSpec((MLP_BLOCK, EMB), lambda i: (i, 0)),
        out_shape=jax.ShapeDtypeStruct((N_ITEMS, EMB), jnp.float32),
    )(x, W1, b1.reshape(1, HIDDEN), W2, b2.reshape(1, EMB))


def _norm_body(raw_ref, out_ref):
    u = raw_ref[...]
    nrm = jnp.sqrt(jnp.sum(u * u, axis=-1, keepdims=True))
    out_ref[...] = u / jnp.maximum(nrm, 1e-12)


def _norm(raw):
    return pl.pallas_call(
        _norm_body,
        grid=(N_USERS // MLP_BLOCK,),
        in_specs=[pl.BlockSpec((MLP_BLOCK, EMB), lambda i: (i, 0))],
        out_specs=pl.BlockSpec((MLP_BLOCK, EMB), lambda i: (i, 0)),
        out_shape=jax.ShapeDtypeStruct((N_USERS, EMB), jnp.float32),
    )(raw)


@functools.partial(
    pl.kernel,
    out_type=jax.ShapeDtypeStruct((N_USERS, EMB), jnp.float32),
    mesh=plsc.VectorSubcoreMesh(core_axis_name="c", subcore_axis_name="s",
                                num_cores=NC, num_subcores=NS),
    compiler_params=pltpu.CompilerParams(use_tc_tiling_on_sc=False),
    scratch_types=[
        pltpu.VMEM((CHUNK,), jnp.int32),
        pltpu.VMEM((CHUNK,), jnp.int32),
        pltpu.VMEM((CHUNK,), jnp.int32),
        pltpu.VMEM((CHUNK, EMB), jnp.float32),
        pltpu.VMEM((CHUNK, EMB), jnp.float32),
        pltpu.VMEM((16,), jnp.int32),
        pltpu.VMEM_SHARED((U_SC_PAD, EMB), jnp.float32),
        pltpu.SemaphoreType.DMA,
    ],
)
def _spmm(fn_hbm, row_hbm, col_hbm, es_hbm, z64_hbm, raw_hbm,
          colv, rowv, locv, rows, zrow, esv, acc, sem):
    c = lax.axis_index("c")
    s = lax.axis_index("s")

    pltpu.sync_copy(z64_hbm, zrow)
    pltpu.sync_copy(es_hbm, esv)
    es = esv[...][0]

    z0 = pl.multiple_of(s * U_TILE, 8)
    for j in range(U_TILE // CHUNK):
        pltpu.sync_copy(zrow, acc.at[pl.ds(z0 + j * CHUNK, CHUNK)])
    tail = U_TILE - (U_TILE // CHUNK) * CHUNK
    if tail:
        pltpu.sync_copy(zrow.at[pl.ds(0, tail)],
                        acc.at[pl.ds(z0 + (U_TILE // CHUNK) * CHUNK, tail)])
    plsc.subcore_barrier()

    len_c = jnp.where(c == 0, es, N_EDGES - es)
    base_e = jnp.where(c == 0, 0, es)
    base_u = c * U_SC
    start = base_e + ((s * len_c) >> 4)
    end = base_e + (((s + 1) * len_c) >> 4)
    a0 = pl.multiple_of((start >> 3) << 3, 8)
    nch = (end - a0 + (CHUNK - 1)) >> 7

    def body(k, carry):
        pos = pl.multiple_of(a0 + k * CHUNK, 8)
        pltpu.sync_copy(row_hbm.at[pl.ds(pos, CHUNK)], rowv)
        pltpu.sync_copy(col_hbm.at[pl.ds(pos, CHUNK)], colv)
        for g in range(CHUNK // 16):
            r = rowv[pl.ds(g * 16, 16)]
            gidx = pos + g * 16 + lax.iota(jnp.int32, 16)
            valid = (gidx >= start) & (gidx < end)
            locv[pl.ds(g * 16, 16)] = jnp.where(valid, r - base_u, TRASH)
        pltpu.async_copy(fn_hbm.at[colv], rows, sem).wait()
        pltpu.sync_copy(rows, acc.at[locv], add=True)
        return carry

    lax.fori_loop(0, nch, body, 0)
    plsc.subcore_barrier()

    u0 = pl.multiple_of(s * U_TILE, 8)
    w0 = pl.multiple_of(base_u + u0, 8)

    @pl.when(s < NS - 1)
    def _():
        pltpu.sync_copy(acc.at[pl.ds(u0, U_TILE)],
                        raw_hbm.at[pl.ds(w0, U_TILE)])

    @pl.when(s == NS - 1)
    def _():
        pltpu.sync_copy(acc.at[pl.ds(u0, U_LAST)],
                        raw_hbm.at[pl.ds(w0, U_LAST)])


def kernel(teacher_input, W1, b1, W2, b2, edge_row, edge_col):
    fn = _mlp(teacher_input, W1, b1, W2, b2)
    es = jnp.searchsorted(edge_row, U_SC).astype(jnp.int32)
    es_vec = jnp.full((16,), es, dtype=jnp.int32)
    row_pad = jnp.concatenate([edge_row, jnp.zeros((CHUNK,), jnp.int32)])
    col_pad = jnp.concatenate([edge_col, jnp.zeros((CHUNK,), jnp.int32)])
    z64 = jnp.zeros((CHUNK, EMB), jnp.float32)
    raw = _spmm(fn, row_pad, col_pad, es_vec, z64)
    return (_norm(raw), fn)


# speedup vs baseline: 19.3370x; 1.2722x over previous
import functools

import jax
import jax.numpy as jnp
from jax import lax
from jax.experimental import pallas as pl
from jax.experimental.pallas import tpu as pltpu
from jax.experimental.pallas import tpu_sc as plsc

N_USERS = 50000
N_ITEMS = 50000
N_EDGES = 800000
TEACHER_DIM = 256
HIDDEN = 128
EMB = 64

NC = 2
NS = 16
U_SC = N_USERS // NC
U_TILE = 1568
U_SC_PAD = NS * U_TILE
TRASH = U_SC
CHUNK = 128
NBUF = 2
E_PAD = N_EDGES + NBUF * CHUNK
E_ROWS = E_PAD // CHUNK
U_LAST = U_SC - (NS - 1) * U_TILE

MLP_BLOCK = 1000


def _mlp_body(x_ref, w1_ref, b1_ref, w2_ref, b2_ref, fn_ref):
    h = jnp.maximum(
        jnp.dot(x_ref[...], w1_ref[...], preferred_element_type=jnp.float32)
        + b1_ref[...], 0.0)
    f = jnp.dot(h, w2_ref[...], preferred_element_type=jnp.float32) + b2_ref[...]
    nrm = jnp.sqrt(jnp.sum(f * f, axis=-1, keepdims=True))
    fn_ref[...] = f / jnp.maximum(nrm, 1e-12)


def _mlp(x, W1, b1, W2, b2):
    return pl.pallas_call(
        _mlp_body,
        grid=(N_ITEMS // MLP_BLOCK,),
        in_specs=[
            pl.BlockSpec((MLP_BLOCK, TEACHER_DIM), lambda i: (i, 0)),
            pl.BlockSpec((TEACHER_DIM, HIDDEN), lambda i: (0, 0)),
            pl.BlockSpec((1, HIDDEN), lambda i: (0, 0)),
            pl.BlockSpec((HIDDEN, EMB), lambda i: (0, 0)),
            pl.BlockSpec((1, EMB), lambda i: (0, 0)),
        ],
        out_specs=pl.BlockSpec((MLP_BLOCK, EMB), lambda i: (i, 0)),
        out_shape=jax.ShapeDtypeStruct((N_ITEMS, EMB), jnp.float32),
    )(x, W1, b1.reshape(1, HIDDEN), W2, b2.reshape(1, EMB))


def _norm_body(raw_ref, out_ref):
    u = raw_ref[...]
    nrm = jnp.sqrt(jnp.sum(u * u, axis=-1, keepdims=True))
    out_ref[...] = u / jnp.maximum(nrm, 1e-12)


def _norm(raw):
    return pl.pallas_call(
        _norm_body,
        grid=(N_USERS // MLP_BLOCK,),
        in_specs=[pl.BlockSpec((MLP_BLOCK, EMB), lambda i: (i, 0))],
        out_specs=pl.BlockSpec((MLP_BLOCK, EMB), lambda i: (i, 0)),
        out_shape=jax.ShapeDtypeStruct((N_USERS, EMB), jnp.float32),
    )(raw)


@functools.partial(
    pl.kernel,
    out_type=jax.ShapeDtypeStruct((N_USERS, EMB), jnp.float32),
    mesh=plsc.VectorSubcoreMesh(core_axis_name="c", subcore_axis_name="s",
                                num_cores=NC, num_subcores=NS),
    compiler_params=pltpu.CompilerParams(use_tc_tiling_on_sc=False),
    scratch_types=[
        pltpu.VMEM((NBUF, CHUNK), jnp.int32),
        pltpu.VMEM((NBUF, CHUNK), jnp.int32),
        pltpu.VMEM((NBUF, CHUNK), jnp.int32),
        pltpu.VMEM((NBUF, CHUNK, EMB), jnp.float32),
        pltpu.VMEM((CHUNK, EMB), jnp.float32),
        pltpu.VMEM((16,), jnp.int32),
        pltpu.VMEM_SHARED((U_SC_PAD, EMB), jnp.float32),
        pltpu.SemaphoreType.DMA,
        pltpu.SemaphoreType.DMA,
    ],
)
def _spmm(fn_hbm, row_hbm, col_hbm, es_hbm, z64_hbm, raw_hbm,
          colv, rowv, locv, rows, zrow, esv, acc, sem0, sem1):
    c = lax.axis_index("c")
    s = lax.axis_index("s")
    sems = (sem0, sem1)

    pltpu.sync_copy(z64_hbm, zrow)
    pltpu.sync_copy(es_hbm, esv)
    es = esv[...][0]

    z0 = pl.multiple_of(s * U_TILE, 8)
    for j in range(U_TILE // CHUNK):
        pltpu.sync_copy(zrow, acc.at[pl.ds(z0 + j * CHUNK, CHUNK)])
    tail = U_TILE - (U_TILE // CHUNK) * CHUNK
    if tail:
        pltpu.sync_copy(zrow.at[pl.ds(0, tail)],
                        acc.at[pl.ds(z0 + (U_TILE // CHUNK) * CHUNK, tail)])
    plsc.subcore_barrier()

    len_c = jnp.where(c == 0, es, N_EDGES - es)
    base_e = jnp.where(c == 0, 0, es)
    base_u = c * U_SC
    start = base_e + ((s * len_c) >> 4)
    end = base_e + (((s + 1) * len_c) >> 4)
    c0 = start >> 7
    nch = (end - (c0 << 7) + (CHUNK - 1)) >> 7

    def compute_loc(b, pos):
        for g in range(CHUNK // 16):
            r = rowv[b, pl.ds(g * 16, 16)]
            gidx = pos + g * 16 + lax.iota(jnp.int32, 16)
            valid = (gidx >= start) & (gidx < end)
            locv[b, pl.ds(g * 16, 16)] = jnp.where(valid, r - base_u, TRASH)

    def body(kk, carry):
        i0 = c0 + kk * NBUF
        pltpu.sync_copy(row_hbm.at[pl.ds(i0, NBUF)], rowv)
        pltpu.sync_copy(col_hbm.at[pl.ds(i0, NBUF)], colv)
        live1 = kk * NBUF + 1 < nch
        compute_loc(0, (i0 << 7))
        cp0 = pltpu.async_copy(fn_hbm.at[colv.at[0]], rows.at[0], sems[0])

        @pl.when(live1)
        def _():
            compute_loc(1, (i0 << 7) + CHUNK)
            cp1 = pltpu.async_copy(fn_hbm.at[colv.at[1]], rows.at[1], sems[1])
            del cp1

        cp0.wait()
        pltpu.sync_copy(rows.at[0], acc.at[locv.at[0]], add=True)

        @pl.when(live1)
        def _():
            pltpu.make_async_copy(fn_hbm.at[colv.at[1]], rows.at[1],
                                  sems[1]).wait()
            pltpu.sync_copy(rows.at[1], acc.at[locv.at[1]], add=True)

        return carry

    lax.fori_loop(0, (nch + NBUF - 1) >> 1, body, 0)
    plsc.subcore_barrier()

    u0 = pl.multiple_of(s * U_TILE, 8)
    w0 = pl.multiple_of(base_u + u0, 8)

    @pl.when(s < NS - 1)
    def _():
        pltpu.sync_copy(acc.at[pl.ds(u0, U_TILE)],
                        raw_hbm.at[pl.ds(w0, U_TILE)])

    @pl.when(s == NS - 1)
    def _():
        pltpu.sync_copy(acc.at[pl.ds(u0, U_LAST)],
                        raw_hbm.at[pl.ds(w0, U_LAST)])


def kernel(teacher_input, W1, b1, W2, b2, edge_row, edge_col):
    fn = _mlp(teacher_input, W1, b1, W2, b2)
    es = jnp.searchsorted(edge_row, U_SC).astype(jnp.int32)
    es_vec = jnp.full((16,), es, dtype=jnp.int32)
    pad = jnp.zeros((NBUF * CHUNK,), jnp.int32)
    row2d = jnp.concatenate([edge_row, pad]).reshape(E_ROWS, CHUNK)
    col2d = jnp.concatenate([edge_col, pad]).reshape(E_ROWS, CHUNK)
    z64 = jnp.zeros((CHUNK, EMB), jnp.float32)
    raw = _spmm(fn, row2d, col2d, es_vec, z64)
    return (_norm(raw), fn)


# speedup vs baseline: 22.6144x; 1.1695x over previous
import functools

import jax
import jax.numpy as jnp
from jax import lax
from jax.experimental import pallas as pl
from jax.experimental.pallas import tpu as pltpu
from jax.experimental.pallas import tpu_sc as plsc

N_USERS = 50000
N_ITEMS = 50000
N_EDGES = 800000
TEACHER_DIM = 256
HIDDEN = 128
EMB = 64

NC = 2
NS = 16
NPASS = 2
CUTS_U = (0, 12504, 25000, 37504, 50000)
U_TILE = 784
U_ACC = NS * U_TILE
CHUNK = 128
NBUF = 4
E_PAD = N_EDGES + NBUF * CHUNK
E_ROWS = E_PAD // CHUNK

MLP_BLOCK = 1000


def _mlp_body(x_ref, w1_ref, b1_ref, w2_ref, b2_ref, fn_ref):
    h = jnp.maximum(
        jnp.dot(x_ref[...], w1_ref[...], preferred_element_type=jnp.float32)
        + b1_ref[...], 0.0)
    f = jnp.dot(h, w2_ref[...], preferred_element_type=jnp.float32) + b2_ref[...]
    nrm = jnp.sqrt(jnp.sum(f * f, axis=-1, keepdims=True))
    fn_ref[...] = f / jnp.maximum(nrm, 1e-12)


def _mlp(x, W1, b1, W2, b2):
    return pl.pallas_call(
        _mlp_body,
        grid=(N_ITEMS // MLP_BLOCK,),
        in_specs=[
            pl.BlockSpec((MLP_BLOCK, TEACHER_DIM), lambda i: (i, 0)),
            pl.BlockSpec((TEACHER_DIM, HIDDEN), lambda i: (0, 0)),
            pl.BlockSpec((1, HIDDEN), lambda i: (0, 0)),
            pl.BlockSpec((HIDDEN, EMB), lambda i: (0, 0)),
            pl.BlockSpec((1, EMB), lambda i: (0, 0)),
        ],
        out_specs=pl.BlockSpec((MLP_BLOCK, EMB), lambda i: (i, 0)),
        out_shape=jax.ShapeDtypeStruct((N_ITEMS, EMB), jnp.float32),
    )(x, W1, b1.reshape(1, HIDDEN), W2, b2.reshape(1, EMB))


def _norm_body(raw_ref, out_ref):
    u = raw_ref[...]
    nrm = jnp.sqrt(jnp.sum(u * u, axis=-1, keepdims=True))
    out_ref[...] = u / jnp.maximum(nrm, 1e-12)


def _norm(raw):
    return pl.pallas_call(
        _norm_body,
        grid=(N_USERS // MLP_BLOCK,),
        in_specs=[pl.BlockSpec((MLP_BLOCK, EMB), lambda i: (i, 0))],
        out_specs=pl.BlockSpec((MLP_BLOCK, EMB), lambda i: (i, 0)),
        out_shape=jax.ShapeDtypeStruct((N_USERS, EMB), jnp.float32),
    )(raw)


@functools.partial(
    pl.kernel,
    out_type=jax.ShapeDtypeStruct((N_USERS, EMB), jnp.float32),
    mesh=plsc.VectorSubcoreMesh(core_axis_name="c", subcore_axis_name="s",
                                num_cores=NC, num_subcores=NS),
    compiler_params=pltpu.CompilerParams(use_tc_tiling_on_sc=False),
    scratch_types=[
        pltpu.VMEM((NBUF, CHUNK), jnp.int32),
        pltpu.VMEM((NBUF, CHUNK), jnp.int32),
        pltpu.VMEM((NBUF, CHUNK), jnp.int32),
        pltpu.VMEM((NBUF, CHUNK, EMB), jnp.float32),
        pltpu.VMEM((CHUNK, EMB), jnp.float32),
        pltpu.VMEM((16,), jnp.int32),
        pltpu.VMEM_SHARED((U_ACC, EMB), jnp.float32),
        pltpu.SemaphoreType.DMA,
        pltpu.SemaphoreType.DMA,
        pltpu.SemaphoreType.DMA,
        pltpu.SemaphoreType.DMA,
    ],
)
def _spmm(fn_hbm, row_hbm, col_hbm, cuts_hbm, z64_hbm, raw_hbm,
          colv, rowv, locv, rows, zrow, cutsv, acc, *sems):
    c = lax.axis_index("c")
    s = lax.axis_index("s")

    pltpu.sync_copy(z64_hbm, zrow)
    pltpu.sync_copy(cuts_hbm, cutsv)
    cv = cutsv[...]

    for p in range(NPASS):
        lo = jnp.where(c == 0, cv[p], cv[NPASS + p])
        hi = jnp.where(c == 0, cv[p + 1], cv[NPASS + p + 1])
        base_u = jnp.where(c == 0, CUTS_U[p], CUTS_U[NPASS + p])
        seg_users = (CUTS_U[p + 1] - CUTS_U[p] if p == 0
                     else CUTS_U[NPASS + p + 1] - CUTS_U[NPASS + p])
        trash = seg_users

        z0 = pl.multiple_of(s * U_TILE, 8)
        for j in range(U_TILE // CHUNK):
            pltpu.sync_copy(zrow, acc.at[pl.ds(z0 + j * CHUNK, CHUNK)])
        tail = U_TILE - (U_TILE // CHUNK) * CHUNK
        if tail:
            pltpu.sync_copy(zrow.at[pl.ds(0, tail)],
                            acc.at[pl.ds(z0 + (U_TILE // CHUNK) * CHUNK,
                                         tail)])
        plsc.subcore_barrier()

        length = hi - lo
        start = lo + ((s * length) >> 4)
        end = lo + (((s + 1) * length) >> 4)
        c0 = start >> 7
        nch = (end - (c0 << 7) + (CHUNK - 1)) >> 7

        def compute_loc(b, pos, start=start, end=end, base_u=base_u,
                        trash=trash):
            for g in range(CHUNK // 16):
                r = rowv[b, pl.ds(g * 16, 16)]
                gidx = pos + g * 16 + lax.iota(jnp.int32, 16)
                valid = (gidx >= start) & (gidx < end)
                locv[b, pl.ds(g * 16, 16)] = jnp.where(valid, r - base_u,
                                                       trash)

        def body(kk, carry, c0=c0, nch=nch, compute_loc=compute_loc):
            i0 = c0 + kk * NBUF
            pltpu.sync_copy(row_hbm.at[pl.ds(i0, NBUF)], rowv)
            pltpu.sync_copy(col_hbm.at[pl.ds(i0, NBUF)], colv)
            compute_loc(0, (i0 << 7))
            pltpu.async_copy(fn_hbm.at[colv.at[0]], rows.at[0], sems[0])
            for b in range(1, NBUF):
                @pl.when(kk * NBUF + b < nch)
                def _(b=b):
                    compute_loc(b, (i0 << 7) + b * CHUNK)
                    pltpu.async_copy(fn_hbm.at[colv.at[b]], rows.at[b],
                                     sems[b])

            pltpu.make_async_copy(fn_hbm.at[colv.at[0]], rows.at[0],
                                  sems[0]).wait()
            pltpu.sync_copy(rows.at[0], acc.at[locv.at[0]], add=True)
            for b in range(1, NBUF):
                @pl.when(kk * NBUF + b < nch)
                def _(b=b):
                    pltpu.make_async_copy(fn_hbm.at[colv.at[b]], rows.at[b],
                                          sems[b]).wait()
                    pltpu.sync_copy(rows.at[b], acc.at[locv.at[b]], add=True)

            return carry

        lax.fori_loop(0, (nch + NBUF - 1) >> 2, body, 0)
        plsc.subcore_barrier()

        u0 = pl.multiple_of(s * U_TILE, 8)
        w0 = pl.multiple_of(base_u + u0, 8)
        last = seg_users - (NS - 1) * U_TILE

        @pl.when(s < NS - 1)
        def _(u0=u0, w0=w0):
            pltpu.sync_copy(acc.at[pl.ds(u0, U_TILE)],
                            raw_hbm.at[pl.ds(w0, U_TILE)])

        @pl.when(s == NS - 1)
        def _(u0=u0, w0=w0, last=last):
            pltpu.sync_copy(acc.at[pl.ds(u0, last)],
                            raw_hbm.at[pl.ds(w0, last)])


def kernel(teacher_input, W1, b1, W2, b2, edge_row, edge_col):
    fn = _mlp(teacher_input, W1, b1, W2, b2)
    cuts = jnp.searchsorted(
        edge_row, jnp.array(CUTS_U[1:4], dtype=jnp.int32)).astype(jnp.int32)
    cuts_vec = jnp.zeros((16,), jnp.int32)
    cuts_vec = cuts_vec.at[1:4].set(cuts).at[4].set(N_EDGES)
    pad = jnp.zeros((NBUF * CHUNK,), jnp.int32)
    row2d = jnp.concatenate([edge_row, pad]).reshape(E_ROWS, CHUNK)
    col2d = jnp.concatenate([edge_col, pad]).reshape(E_ROWS, CHUNK)
    z64 = jnp.zeros((CHUNK, EMB), jnp.float32)
    raw = _spmm(fn, row2d, col2d, cuts_vec, z64)
    return (_norm(raw), fn)


# speedup vs baseline: 25.3253x; 1.1199x over previous
import functools

import jax
import jax.numpy as jnp
from jax import lax
from jax.experimental import pallas as pl
from jax.experimental.pallas import tpu as pltpu
from jax.experimental.pallas import tpu_sc as plsc

N_USERS = 50000
N_ITEMS = 50000
N_EDGES = 800000
TEACHER_DIM = 256
HIDDEN = 128
EMB = 64

NC = 2
NS = 16
NPASS = 2
CUTS_U = (0, 12504, 25000, 37504, 50000)
U_TILE = 784
U_ACC = NS * U_TILE
CHUNK = 128
NBUF = 6
E_PAD = N_EDGES + NBUF * CHUNK
E_ROWS = E_PAD // CHUNK

MLP_BLOCK = 2000


def _mlp_body(x_ref, w1_ref, b1_ref, w2_ref, b2_ref, fn_ref):
    h = jnp.maximum(
        jnp.dot(x_ref[...], w1_ref[...], preferred_element_type=jnp.float32)
        + b1_ref[...], 0.0)
    f = jnp.dot(h, w2_ref[...], preferred_element_type=jnp.float32) + b2_ref[...]
    nrm = jnp.sqrt(jnp.sum(f * f, axis=-1, keepdims=True))
    fn_ref[...] = f / jnp.maximum(nrm, 1e-12)


def _mlp(x, W1, b1, W2, b2):
    return pl.pallas_call(
        _mlp_body,
        grid=(N_ITEMS // MLP_BLOCK,),
        in_specs=[
            pl.BlockSpec((MLP_BLOCK, TEACHER_DIM), lambda i: (i, 0)),
            pl.BlockSpec((TEACHER_DIM, HIDDEN), lambda i: (0, 0)),
            pl.BlockSpec((1, HIDDEN), lambda i: (0, 0)),
            pl.BlockSpec((HIDDEN, EMB), lambda i: (0, 0)),
            pl.BlockSpec((1, EMB), lambda i: (0, 0)),
        ],
        out_specs=pl.BlockSpec((MLP_BLOCK, EMB), lambda i: (i, 0)),
        out_shape=jax.ShapeDtypeStruct((N_ITEMS, EMB), jnp.float32),
    )(x, W1, b1.reshape(1, HIDDEN), W2, b2.reshape(1, EMB))


def _norm_body(raw_ref, out_ref):
    u = raw_ref[...]
    nrm = jnp.sqrt(jnp.sum(u * u, axis=-1, keepdims=True))
    out_ref[...] = u / jnp.maximum(nrm, 1e-12)


def _norm(raw):
    return pl.pallas_call(
        _norm_body,
        grid=(N_USERS // MLP_BLOCK,),
        in_specs=[pl.BlockSpec((MLP_BLOCK, EMB), lambda i: (i, 0))],
        out_specs=pl.BlockSpec((MLP_BLOCK, EMB), lambda i: (i, 0)),
        out_shape=jax.ShapeDtypeStruct((N_USERS, EMB), jnp.float32),
    )(raw)


@functools.partial(
    pl.kernel,
    out_type=jax.ShapeDtypeStruct((N_USERS, EMB), jnp.float32),
    mesh=plsc.VectorSubcoreMesh(core_axis_name="c", subcore_axis_name="s",
                                num_cores=NC, num_subcores=NS),
    compiler_params=pltpu.CompilerParams(use_tc_tiling_on_sc=False),
    scratch_types=[
        pltpu.VMEM((NBUF, CHUNK), jnp.int32),
        pltpu.VMEM((NBUF, CHUNK), jnp.int32),
        pltpu.VMEM((NBUF, CHUNK), jnp.int32),
        pltpu.VMEM((NBUF, CHUNK, EMB), jnp.float32),
        pltpu.VMEM((CHUNK, EMB), jnp.float32),
        pltpu.VMEM((16,), jnp.int32),
        pltpu.VMEM_SHARED((U_ACC, EMB), jnp.float32),
        pltpu.SemaphoreType.DMA,
        pltpu.SemaphoreType.DMA,
        pltpu.SemaphoreType.DMA,
        pltpu.SemaphoreType.DMA,
        pltpu.SemaphoreType.DMA,
        pltpu.SemaphoreType.DMA,
    ],
)
def _spmm(fn_hbm, row_hbm, col_hbm, cuts_hbm, z64_hbm, raw_hbm,
          colv, rowv, locv, rows, zrow, cutsv, acc, *sems):
    c = lax.axis_index("c")
    s = lax.axis_index("s")

    pltpu.sync_copy(z64_hbm, zrow)
    pltpu.sync_copy(cuts_hbm, cutsv)
    cv = cutsv[...]

    for p in range(NPASS):
        lo = jnp.where(c == 0, cv[p], cv[NPASS + p])
        hi = jnp.where(c == 0, cv[p + 1], cv[NPASS + p + 1])
        base_u = jnp.where(c == 0, CUTS_U[p], CUTS_U[NPASS + p])
        seg_users = (CUTS_U[p + 1] - CUTS_U[p] if p == 0
                     else CUTS_U[NPASS + p + 1] - CUTS_U[NPASS + p])
        trash = seg_users

        z0 = pl.multiple_of(s * U_TILE, 8)
        for j in range(U_TILE // CHUNK):
            pltpu.sync_copy(zrow, acc.at[pl.ds(z0 + j * CHUNK, CHUNK)])
        tail = U_TILE - (U_TILE // CHUNK) * CHUNK
        if tail:
            pltpu.sync_copy(zrow.at[pl.ds(0, tail)],
                            acc.at[pl.ds(z0 + (U_TILE // CHUNK) * CHUNK,
                                         tail)])
        plsc.subcore_barrier()

        length = hi - lo
        start = lo + ((s * length) >> 4)
        end = lo + (((s + 1) * length) >> 4)
        c0 = start >> 7
        nch = (end - (c0 << 7) + (CHUNK - 1)) >> 7

        def compute_loc(b, pos, start=start, end=end, base_u=base_u,
                        trash=trash):
            for g in range(CHUNK // 16):
                r = rowv[b, pl.ds(g * 16, 16)]
                gidx = pos + g * 16 + lax.iota(jnp.int32, 16)
                valid = (gidx >= start) & (gidx < end)
                locv[b, pl.ds(g * 16, 16)] = jnp.where(valid, r - base_u,
                                                       trash)

        def body(kk, carry, c0=c0, nch=nch, compute_loc=compute_loc):
            i0 = c0 + kk * NBUF
            pltpu.sync_copy(row_hbm.at[pl.ds(i0, NBUF)], rowv)
            pltpu.sync_copy(col_hbm.at[pl.ds(i0, NBUF)], colv)
            compute_loc(0, (i0 << 7))
            pltpu.async_copy(fn_hbm.at[colv.at[0]], rows.at[0], sems[0])
            for b in range(1, NBUF):
                @pl.when(kk * NBUF + b < nch)
                def _(b=b):
                    compute_loc(b, (i0 << 7) + b * CHUNK)
                    pltpu.async_copy(fn_hbm.at[colv.at[b]], rows.at[b],
                                     sems[b])

            pltpu.make_async_copy(fn_hbm.at[colv.at[0]], rows.at[0],
                                  sems[0]).wait()
            pltpu.sync_copy(rows.at[0], acc.at[locv.at[0]], add=True)
            for b in range(1, NBUF):
                @pl.when(kk * NBUF + b < nch)
                def _(b=b):
                    pltpu.make_async_copy(fn_hbm.at[colv.at[b]], rows.at[b],
                                          sems[b]).wait()
                    pltpu.sync_copy(rows.at[b], acc.at[locv.at[b]], add=True)

            return carry

        lax.fori_loop(0, (nch + NBUF - 1) // NBUF, body, 0)
        plsc.subcore_barrier()

        u0 = pl.multiple_of(s * U_TILE, 8)
        w0 = pl.multiple_of(base_u + u0, 8)
        last = seg_users - (NS - 1) * U_TILE

        @pl.when(s < NS - 1)
        def _(u0=u0, w0=w0):
            pltpu.sync_copy(acc.at[pl.ds(u0, U_TILE)],
                            raw_hbm.at[pl.ds(w0, U_TILE)])

        @pl.when(s == NS - 1)
        def _(u0=u0, w0=w0, last=last):
            pltpu.sync_copy(acc.at[pl.ds(u0, last)],
                            raw_hbm.at[pl.ds(w0, last)])


def kernel(teacher_input, W1, b1, W2, b2, edge_row, edge_col):
    fn = _mlp(teacher_input, W1, b1, W2, b2)
    cuts = jnp.searchsorted(
        edge_row, jnp.array(CUTS_U[1:4], dtype=jnp.int32)).astype(jnp.int32)
    cuts_vec = jnp.zeros((16,), jnp.int32)
    cuts_vec = cuts_vec.at[1:4].set(cuts).at[4].set(N_EDGES)
    pad = jnp.zeros((NBUF * CHUNK,), jnp.int32)
    row2d = jnp.concatenate([edge_row, pad]).reshape(E_ROWS, CHUNK)
    col2d = jnp.concatenate([edge_col, pad]).reshape(E_ROWS, CHUNK)
    z64 = jnp.zeros((CHUNK, EMB), jnp.float32)
    raw = _spmm(fn, row2d, col2d, cuts_vec, z64)
    return (_norm(raw), fn)
